# baseline (speedup 1.0000x reference)
import functools
import math

import jax
import jax.numpy as jnp
from jax import lax
from jax.experimental import pallas as pl
from jax.experimental.pallas import tpu as pltpu
from jax.experimental.pallas import tpu_sc as plsc

D_MODEL = 64
SCALE = math.sqrt(D_MODEL)
NUM_CORES = 2
NUM_SUBCORES = 16
NUM_WORKERS = NUM_CORES * NUM_SUBCORES
LANES = 16


@functools.partial(jax.jit, static_argnums=(2, 3))
def _embed(x_flat, lut, B, C):
  b_per_w = B // NUM_WORKERS
  n_chunks = b_per_w // C
  mesh = plsc.VectorSubcoreMesh(core_axis_name="c", subcore_axis_name="s")

  @functools.partial(
      pl.kernel,
      out_type=jax.ShapeDtypeStruct((B, D_MODEL), jnp.float32),
      mesh=mesh,
      compiler_params=pltpu.CompilerParams(use_tc_tiling_on_sc=False),
      scratch_types=[
          pltpu.VMEM((b_per_w,), jnp.int32),
          pltpu.VMEM((C, D_MODEL), jnp.float32),
          pltpu.SemaphoreType.DMA,
      ],
  )
  def body(idx_hbm, lut_hbm, out_hbm, idx_v, rows_v, gsem):
    wid = lax.axis_index("s") * NUM_CORES + lax.axis_index("c")
    base = pl.multiple_of(wid * b_per_w, 8)
    pltpu.sync_copy(idx_hbm.at[pl.ds(base, b_per_w)], idx_v)

    def chunk(g, _):
      off = pl.multiple_of(g * C, 8)
      pltpu.async_copy(
          lut_hbm.at[idx_v.at[pl.ds(off, C)]], rows_v, gsem
      ).wait()

      @plsc.parallel_loop(0, C, 1, unroll=4)
      def _(r):
        for j in range(D_MODEL // LANES):
          sl = pl.ds(j * LANES, LANES)
          rows_v[r, sl] = rows_v[r, sl] * SCALE

      pltpu.sync_copy(rows_v, out_hbm.at[pl.ds(base + off, C)])
      return 0

    lax.fori_loop(0, n_chunks, chunk, 0)

  return body(x_flat, lut)


def kernel(x, lut):
  B, T = x.shape
  x_flat = x.reshape(-1).astype(jnp.int32)
  out = _embed(x_flat, lut, B * T, 512)
  return out.reshape(B, T, D_MODEL)


# speedup vs baseline: 1.0636x; 1.0636x over previous
import functools
import math

import jax
import jax.numpy as jnp
from jax import lax
from jax.experimental import pallas as pl
from jax.experimental.pallas import tpu as pltpu
from jax.experimental.pallas import tpu_sc as plsc

D_MODEL = 64
SCALE = math.sqrt(D_MODEL)
NUM_CORES = 2
NUM_SUBCORES = 16
NUM_WORKERS = NUM_CORES * NUM_SUBCORES
LANES = 16


@functools.partial(jax.jit, static_argnums=(2, 3))
def _embed(x_flat, lut, B, C):
  b_per_w = B // NUM_WORKERS
  n_chunks = b_per_w // C
  n_pairs = n_chunks // 2
  mesh = plsc.VectorSubcoreMesh(core_axis_name="c", subcore_axis_name="s")

  @functools.partial(
      pl.kernel,
      out_type=jax.ShapeDtypeStruct((B, D_MODEL), jnp.float32),
      mesh=mesh,
      compiler_params=pltpu.CompilerParams(use_tc_tiling_on_sc=False),
      scratch_types=[
          pltpu.VMEM((b_per_w,), jnp.int32),
          pltpu.VMEM((2, C, D_MODEL), jnp.float32),
          pltpu.SemaphoreType.DMA,
          pltpu.SemaphoreType.DMA,
          pltpu.SemaphoreType.DMA,
          pltpu.SemaphoreType.DMA,
      ],
  )
  def body(idx_hbm, lut_hbm, out_hbm, idx_v, rows_v, gs0, gs1, os0, os1):
    wid = lax.axis_index("s") * NUM_CORES + lax.axis_index("c")
    base = pl.multiple_of(wid * b_per_w, 8)
    pltpu.sync_copy(idx_hbm.at[pl.ds(base, b_per_w)], idx_v)

    def start_gather(g, slot, sem):
      off = pl.multiple_of(g * C, 8)
      pltpu.async_copy(lut_hbm.at[idx_v.at[pl.ds(off, C)]], rows_v.at[slot], sem)

    def wait_gather(slot, sem):
      pltpu.make_async_copy(
          lut_hbm.at[idx_v.at[pl.ds(0, C)]], rows_v.at[slot], sem
      ).wait()

    def scale(slot):
      @plsc.parallel_loop(0, C, 1, unroll=4)
      def _(r):
        for j in range(D_MODEL // LANES):
          sl = pl.ds(j * LANES, LANES)
          rows_v[slot, r, sl] = rows_v[slot, r, sl] * SCALE

    def start_store(g, slot, sem):
      off = pl.multiple_of(g * C, 8)
      return pltpu.async_copy(
          rows_v.at[slot], out_hbm.at[pl.ds(base + off, C)], sem
      )

    start_gather(0, 0, gs0)
    start_gather(1, 1, gs1)

    def pair(i, _):
      g0 = 2 * i
      wait_gather(0, gs0)
      scale(0)
      d0 = start_store(g0, 0, os0)
      wait_gather(1, gs1)
      scale(1)
      d1 = start_store(g0 + 1, 1, os1)
      d0.wait()
      start_gather(g0 + 2, 0, gs0)
      d1.wait()
      start_gather(g0 + 3, 1, gs1)
      return 0

    lax.fori_loop(0, n_pairs - 1, pair, 0)

    wait_gather(0, gs0)
    scale(0)
    dl0 = start_store(n_chunks - 2, 0, os0)
    wait_gather(1, gs1)
    scale(1)
    dl1 = start_store(n_chunks - 1, 1, os1)
    dl0.wait()
    dl1.wait()

  return body(x_flat, lut)


def kernel(x, lut):
  B, T = x.shape
  x_flat = x.reshape(-1).astype(jnp.int32)
  out = _embed(x_flat, lut, B * T, 640)
  return out.reshape(B, T, D_MODEL)
